# Initial kernel scaffold; baseline (speedup 1.0000x reference)
import jax
import jax.numpy as jnp
from jax.experimental import pallas as pl


def kernel(x, edge_index, W0, b0, W1, b1, W2, b2, l1W, l1b, l2W, l2b, mW1, mb1, mW2, mb2, mW3, mb3, parsing0):
    raise NotImplementedError("write your pallas kernel here")


# speedup vs baseline: 2.1727x; 2.1727x over previous
import functools

import jax
import jax.numpy as jnp
from jax import lax
from jax.experimental import pallas as pl
from jax.experimental.pallas import tpu as pltpu

N = 10000
E = 320000
D_IN = 128
HID = 128
OUT = 8

N_PAD = 10240
_NB = 4
_BN = N_PAD // _NB


def _a_body(x_ref, mW1_ref, mb1_ref, mW2_ref, mb2_ref, mW3_ref, mb3_ref,
            pars_ref, W0_ref, pT_ref, pPT_ref, z0_ref):
    xb = x_ref[...]
    h = jnp.maximum(xb @ mW1_ref[...] + mb1_ref[...], 0.0)
    h = jnp.maximum(h @ mW2_ref[...] + mb2_ref[...], 0.0)
    pT = lax.dot_general(mW3_ref[...], h, (((0,), (1,)), ((), ())),
                         preferred_element_type=jnp.float32)
    pT = pT + mb3_ref[...].reshape(OUT, 1)
    parsing = jnp.maximum(2.0 * pars_ref[...], 0.0)
    pPT = lax.dot_general(parsing, pT, (((0,), (0,)), ((), ())),
                          preferred_element_type=jnp.float32)
    pT_ref[...] = pT
    pPT_ref[...] = pPT
    z0_ref[...] = jnp.dot(xb, W0_ref[...], preferred_element_type=jnp.float32)


def _stage_a(x, mW1, mb1, mW2, mb2, mW3, mb3, parsing0, W0):
    full = lambda s: pl.BlockSpec(s, lambda i: (0, 0))
    return pl.pallas_call(
        _a_body,
        grid=(_NB,),
        in_specs=[
            pl.BlockSpec((_BN, D_IN), lambda i: (i, 0)),
            full((D_IN, 512)), full((1, 512)),
            full((512, 64)), full((1, 64)),
            full((64, OUT)), full((1, OUT)),
            full((OUT, OUT)),
            full((D_IN, HID)),
        ],
        out_specs=[
            pl.BlockSpec((OUT, _BN), lambda i: (0, i)),
            pl.BlockSpec((OUT, _BN), lambda i: (0, i)),
            pl.BlockSpec((_BN, HID), lambda i: (i, 0)),
        ],
        out_shape=[
            jax.ShapeDtypeStruct((OUT, N_PAD), jnp.float32),
            jax.ShapeDtypeStruct((OUT, N_PAD), jnp.float32),
            jax.ShapeDtypeStruct((_BN * _NB, HID), jnp.float32),
        ],
    )(x, mW1, mb1.reshape(1, 512), mW2, mb2.reshape(1, 64), mW3,
      mb3.reshape(1, OUT), parsing0, W0)


def _c2_body(s1_ref, cnt_ref, stats_ref, z0_ref, dinv_ref, g0_ref, ab_ref):
    ssum = stats_ref[0]
    ssq = stats_ref[1]
    mean = ssum / E
    var = (ssq - ssum * ssum / E) / (E - 1)
    alpha = jnp.sqrt(1e-4 / var)
    beta = 1.0 - mean * alpha
    s1 = s1_ref[:, 0:1] + s1_ref[:, 1:2]
    cnt = cnt_ref[:, 0:1] + cnt_ref[:, 1:2]
    deg = alpha * s1 + beta * cnt + 1.0
    ok = deg > 0.0
    dinv = jnp.where(ok, lax.rsqrt(jnp.where(ok, deg, 1.0)), 0.0)
    dinv_ref[...] = dinv
    g0_ref[...] = dinv * z0_ref[...]
    lane = lax.broadcasted_iota(jnp.int32, (1, 16), 1)
    ab_ref[...] = jnp.where(lane == 0, alpha,
                            jnp.where(lane == 1, beta, 0.0))


def _stage_c2(s1g, cntg, stats, z0):
    return pl.pallas_call(
        _c2_body,
        grid=(_NB,),
        in_specs=[
            pl.BlockSpec((_BN, 2), lambda i: (i, 0)),
            pl.BlockSpec((_BN, 2), lambda i: (i, 0)),
            pl.BlockSpec(memory_space=pltpu.SMEM),
            pl.BlockSpec((_BN, HID), lambda i: (i, 0)),
        ],
        out_specs=[
            pl.BlockSpec((_BN, 1), lambda i: (i, 0)),
            pl.BlockSpec((_BN, HID), lambda i: (i, 0)),
            pl.BlockSpec((1, 16), lambda i: (0, 0)),
        ],
        out_shape=[
            jax.ShapeDtypeStruct((N_PAD, 1), jnp.float32),
            jax.ShapeDtypeStruct((N_PAD, HID), jnp.float32),
            jax.ShapeDtypeStruct((1, 16), jnp.float32),
        ],
    )(s1g, cntg, stats, z0)


def _e1_body(pa_ref, pb_ref, g0_ref, dinv_ref, b0_ref, W1_ref,
             h0_ref, g1_ref):
    dinv = dinv_ref[...]
    h0 = jnp.maximum(
        dinv * (pa_ref[...] + pb_ref[...] + g0_ref[...]) + b0_ref[...], 0.0)
    h0_ref[...] = h0
    z1 = jnp.dot(h0, W1_ref[...], preferred_element_type=jnp.float32)
    g1_ref[...] = dinv * z1


def _stage_e1(part, g0, dinv, b0, W1):
    return pl.pallas_call(
        _e1_body,
        grid=(_NB,),
        in_specs=[
            pl.BlockSpec((_BN, HID), lambda i: (i, 0)),
            pl.BlockSpec((_BN, HID), lambda i: (i + _NB, 0)),
            pl.BlockSpec((_BN, HID), lambda i: (i, 0)),
            pl.BlockSpec((_BN, 1), lambda i: (i, 0)),
            pl.BlockSpec((1, HID), lambda i: (0, 0)),
            pl.BlockSpec((HID, HID), lambda i: (0, 0)),
        ],
        out_specs=[
            pl.BlockSpec((_BN, HID), lambda i: (i, 0)),
            pl.BlockSpec((_BN, HID), lambda i: (i, 0)),
        ],
        out_shape=[
            jax.ShapeDtypeStruct((N_PAD, HID), jnp.float32),
            jax.ShapeDtypeStruct((N_PAD, HID), jnp.float32),
        ],
    )(part, part, g0, dinv, b0.reshape(1, HID), W1)


def _e2_body(pa_ref, pb_ref, g1_ref, dinv_ref, b1_ref, h0_ref,
             l1a_ref, l1b_ref, l1bias_ref, l2W_ref, l2b_ref, o_ref):
    dinv = dinv_ref[...]
    h1 = jnp.maximum(
        dinv * (pa_ref[...] + pb_ref[...] + g1_ref[...]) + b1_ref[...], 0.0)
    j = (jnp.dot(h0_ref[...], l1a_ref[...], preferred_element_type=jnp.float32)
         + jnp.dot(h1, l1b_ref[...], preferred_element_type=jnp.float32)
         + l1bias_ref[...])
    j = jnp.maximum(j, 0.0)
    o_ref[...] = (jnp.dot(j, l2W_ref[...], preferred_element_type=jnp.float32)
                  + l2b_ref[...])


def _stage_e2(part, g1, dinv, b1, h0, l1W, l1b, l2W, l2b):
    return pl.pallas_call(
        _e2_body,
        grid=(_NB,),
        in_specs=[
            pl.BlockSpec((_BN, HID), lambda i: (i, 0)),
            pl.BlockSpec((_BN, HID), lambda i: (i + _NB, 0)),
            pl.BlockSpec((_BN, HID), lambda i: (i, 0)),
            pl.BlockSpec((_BN, 1), lambda i: (i, 0)),
            pl.BlockSpec((1, HID), lambda i: (0, 0)),
            pl.BlockSpec((_BN, HID), lambda i: (i, 0)),
            pl.BlockSpec((HID, HID), lambda i: (0, 0)),
            pl.BlockSpec((HID, HID), lambda i: (0, 0)),
            pl.BlockSpec((1, HID), lambda i: (0, 0)),
            pl.BlockSpec((HID, OUT), lambda i: (0, 0)),
            pl.BlockSpec((1, OUT), lambda i: (0, 0)),
        ],
        out_specs=pl.BlockSpec((_BN, OUT), lambda i: (i, 0)),
        out_shape=jax.ShapeDtypeStruct((N_PAD, OUT), jnp.float32),
    )(part, part, g1, dinv, b1.reshape(1, HID), h0, l1W[:HID], l1W[HID:],
      l1b.reshape(1, HID), l2W, l2b.reshape(1, OUT))


def _edge_weights_jax(pT, pPT, row, col):
    w = jnp.sum(pT[:, row] * pPT[:, col], axis=0)
    ssum = jnp.sum(w)
    ssq = jnp.sum(w * w)
    s1 = jax.ops.segment_sum(w, col, num_segments=N)
    cnt = jax.ops.segment_sum(jnp.ones((E,), jnp.float32), col, num_segments=N)
    return w, jnp.stack([ssum, ssq]), s1, cnt


def _conv_scatter_jax(g, row, col, ew):
    acc = jax.ops.segment_sum(ew[:, None] * g[row], col, num_segments=N_PAD)
    return jnp.concatenate([acc, jnp.zeros_like(acc)], axis=0)


def kernel(x, edge_index, W0, b0, W1, b1, W2, b2, l1W, l1b, l2W, l2b,
           mW1, mb1, mW2, mb2, mW3, mb3, parsing0):
    row, col = edge_index[0], edge_index[1]
    xp = jnp.pad(x, ((0, N_PAD - N), (0, 0)))

    pT, pPT, z0 = _stage_a(xp, mW1, mb1, mW2, mb2, mW3, mb3, parsing0, W0)

    w, stats, s1, cnt = _edge_weights_jax(pT, pPT, row, col)
    zpad = jnp.zeros((N_PAD - N,), jnp.float32)
    s1g = jnp.stack([jnp.concatenate([s1, zpad]),
                     jnp.zeros((N_PAD,), jnp.float32)], axis=1)
    cntg = jnp.stack([jnp.concatenate([cnt, zpad]),
                      jnp.zeros((N_PAD,), jnp.float32)], axis=1)

    dinv, g0, ab = _stage_c2(s1g, cntg, stats, z0)
    alpha, beta = ab[0, 0], ab[0, 1]
    ew = alpha * w + beta

    part0 = _conv_scatter_jax(g0, row, col, ew)
    h0, g1 = _stage_e1(part0, g0, dinv, b0, W1)
    part1 = _conv_scatter_jax(g1, row, col, ew)
    o = _stage_e2(part1, g1, dinv, b1, h0, l1W, l1b, l2W, l2b)
    return o[:N]


# speedup vs baseline: 4.3287x; 1.9923x over previous
import functools

import jax
import jax.numpy as jnp
from jax import lax
from jax.experimental import pallas as pl
from jax.experimental.pallas import tpu as pltpu
from jax.experimental.pallas import tpu_sc as plsc

N = 10000
E = 320000
D_IN = 128
HID = 128
OUT = 8

N_PAD = 10240
_NB = 4
_BN = N_PAD // _NB


def _a_body(x_ref, mW1_ref, mb1_ref, mW2_ref, mb2_ref, mW3_ref, mb3_ref,
            pars_ref, W0_ref, pT_ref, pPT_ref, z0_ref):
    xb = x_ref[...]
    h = jnp.maximum(xb @ mW1_ref[...] + mb1_ref[...], 0.0)
    h = jnp.maximum(h @ mW2_ref[...] + mb2_ref[...], 0.0)
    pT = lax.dot_general(mW3_ref[...], h, (((0,), (1,)), ((), ())),
                         preferred_element_type=jnp.float32)
    pT = pT + mb3_ref[...].reshape(OUT, 1)
    parsing = jnp.maximum(2.0 * pars_ref[...], 0.0)
    pPT = lax.dot_general(parsing, pT, (((0,), (0,)), ((), ())),
                          preferred_element_type=jnp.float32)
    pT_ref[...] = pT
    pPT_ref[...] = pPT
    z0_ref[...] = jnp.dot(xb, W0_ref[...], preferred_element_type=jnp.float32)


def _stage_a(x, mW1, mb1, mW2, mb2, mW3, mb3, parsing0, W0):
    full = lambda s: pl.BlockSpec(s, lambda i: (0, 0))
    return pl.pallas_call(
        _a_body,
        grid=(_NB,),
        in_specs=[
            pl.BlockSpec((_BN, D_IN), lambda i: (i, 0)),
            full((D_IN, 512)), full((1, 512)),
            full((512, 64)), full((1, 64)),
            full((64, OUT)), full((1, OUT)),
            full((OUT, OUT)),
            full((D_IN, HID)),
        ],
        out_specs=[
            pl.BlockSpec((OUT, _BN), lambda i: (0, i)),
            pl.BlockSpec((OUT, _BN), lambda i: (0, i)),
            pl.BlockSpec((_BN, HID), lambda i: (i, 0)),
        ],
        out_shape=[
            jax.ShapeDtypeStruct((OUT, N_PAD), jnp.float32),
            jax.ShapeDtypeStruct((OUT, N_PAD), jnp.float32),
            jax.ShapeDtypeStruct((_BN * _NB, HID), jnp.float32),
        ],
    )(x, mW1, mb1.reshape(1, 512), mW2, mb2.reshape(1, 64), mW3,
      mb3.reshape(1, OUT), parsing0, W0)


def _c2_body(s1_ref, cnt_ref, stats_ref, z0_ref, dinv_ref, g0_ref, ab_ref):
    ssum = stats_ref[0]
    ssq = stats_ref[1]
    mean = ssum / E
    var = (ssq - ssum * ssum / E) / (E - 1)
    alpha = jnp.sqrt(1e-4 / var)
    beta = 1.0 - mean * alpha
    s1 = s1_ref[:, 0:1] + s1_ref[:, 1:2]
    cnt = cnt_ref[:, 0:1] + cnt_ref[:, 1:2]
    deg = alpha * s1 + beta * cnt + 1.0
    ok = deg > 0.0
    dinv = jnp.where(ok, lax.rsqrt(jnp.where(ok, deg, 1.0)), 0.0)
    dinv_ref[...] = dinv
    g0_ref[...] = dinv * z0_ref[...]
    lane = lax.broadcasted_iota(jnp.int32, (1, 16), 1)
    ab_ref[...] = jnp.where(lane == 0, alpha,
                            jnp.where(lane == 1, beta, 0.0))


def _stage_c2(s1g, cntg, stats, z0):
    return pl.pallas_call(
        _c2_body,
        grid=(_NB,),
        in_specs=[
            pl.BlockSpec((_BN, 2), lambda i: (i, 0)),
            pl.BlockSpec((_BN, 2), lambda i: (i, 0)),
            pl.BlockSpec(memory_space=pltpu.SMEM),
            pl.BlockSpec((_BN, HID), lambda i: (i, 0)),
        ],
        out_specs=[
            pl.BlockSpec((_BN, 1), lambda i: (i, 0)),
            pl.BlockSpec((_BN, HID), lambda i: (i, 0)),
            pl.BlockSpec((1, 16), lambda i: (0, 0)),
        ],
        out_shape=[
            jax.ShapeDtypeStruct((N_PAD, 1), jnp.float32),
            jax.ShapeDtypeStruct((N_PAD, HID), jnp.float32),
            jax.ShapeDtypeStruct((1, 16), jnp.float32),
        ],
    )(s1g, cntg, stats, z0)


def _e1_body(pa_ref, pb_ref, g0_ref, dinv_ref, b0_ref, W1_ref,
             h0_ref, g1_ref):
    dinv = dinv_ref[...]
    h0 = jnp.maximum(
        dinv * (pa_ref[...] + pb_ref[...] + g0_ref[...]) + b0_ref[...], 0.0)
    h0_ref[...] = h0
    z1 = jnp.dot(h0, W1_ref[...], preferred_element_type=jnp.float32)
    g1_ref[...] = dinv * z1


def _stage_e1(part, g0, dinv, b0, W1):
    return pl.pallas_call(
        _e1_body,
        grid=(_NB,),
        in_specs=[
            pl.BlockSpec((_BN, HID), lambda i: (i, 0)),
            pl.BlockSpec((_BN, HID), lambda i: (i + _NB, 0)),
            pl.BlockSpec((_BN, HID), lambda i: (i, 0)),
            pl.BlockSpec((_BN, 1), lambda i: (i, 0)),
            pl.BlockSpec((1, HID), lambda i: (0, 0)),
            pl.BlockSpec((HID, HID), lambda i: (0, 0)),
        ],
        out_specs=[
            pl.BlockSpec((_BN, HID), lambda i: (i, 0)),
            pl.BlockSpec((_BN, HID), lambda i: (i, 0)),
        ],
        out_shape=[
            jax.ShapeDtypeStruct((N_PAD, HID), jnp.float32),
            jax.ShapeDtypeStruct((N_PAD, HID), jnp.float32),
        ],
    )(part, part, g0, dinv, b0.reshape(1, HID), W1)


def _e2_body(pa_ref, pb_ref, g1_ref, dinv_ref, b1_ref, h0_ref,
             l1a_ref, l1b_ref, l1bias_ref, l2W_ref, l2b_ref, o_ref):
    dinv = dinv_ref[...]
    h1 = jnp.maximum(
        dinv * (pa_ref[...] + pb_ref[...] + g1_ref[...]) + b1_ref[...], 0.0)
    j = (jnp.dot(h0_ref[...], l1a_ref[...], preferred_element_type=jnp.float32)
         + jnp.dot(h1, l1b_ref[...], preferred_element_type=jnp.float32)
         + l1bias_ref[...])
    j = jnp.maximum(j, 0.0)
    o_ref[...] = (jnp.dot(j, l2W_ref[...], preferred_element_type=jnp.float32)
                  + l2b_ref[...])


def _stage_e2(part, g1, dinv, b1, h0, l1W, l1b, l2W, l2b):
    return pl.pallas_call(
        _e2_body,
        grid=(_NB,),
        in_specs=[
            pl.BlockSpec((_BN, HID), lambda i: (i, 0)),
            pl.BlockSpec((_BN, HID), lambda i: (i + _NB, 0)),
            pl.BlockSpec((_BN, HID), lambda i: (i, 0)),
            pl.BlockSpec((_BN, 1), lambda i: (i, 0)),
            pl.BlockSpec((1, HID), lambda i: (0, 0)),
            pl.BlockSpec((_BN, HID), lambda i: (i, 0)),
            pl.BlockSpec((HID, HID), lambda i: (0, 0)),
            pl.BlockSpec((HID, HID), lambda i: (0, 0)),
            pl.BlockSpec((1, HID), lambda i: (0, 0)),
            pl.BlockSpec((HID, OUT), lambda i: (0, 0)),
            pl.BlockSpec((1, OUT), lambda i: (0, 0)),
        ],
        out_specs=pl.BlockSpec((_BN, OUT), lambda i: (i, 0)),
        out_shape=jax.ShapeDtypeStruct((N_PAD, OUT), jnp.float32),
    )(part, part, g1, dinv, b1.reshape(1, HID), h0, l1W[:HID], l1W[HID:],
      l1b.reshape(1, HID), l2W, l2b.reshape(1, OUT))


_NW = 32
_EPT = E // _NW
_DCH = 128
_DNCH = 80
_EPAD = _DNCH * _DCH - _EPT
_RPT = N_PAD // 16


def _d_body(g_ref, row_ref, col_ref, ew_ref, zeros_ref, out_ref,
            rowv, colv, ewv, buf, acc, gsem):
    cid = lax.axis_index("c")
    sid = lax.axis_index("s")
    wid = sid * 2 + cid
    base = wid * _DNCH
    pltpu.sync_copy(row_ref.at[pl.ds(base, _DNCH)], rowv)
    pltpu.sync_copy(col_ref.at[pl.ds(base, _DNCH)], colv)
    pltpu.sync_copy(ew_ref.at[pl.ds(base, _DNCH)], ewv)

    pltpu.sync_copy(zeros_ref, acc.at[pl.ds(sid * _RPT, _RPT)])
    plsc.subcore_barrier()

    def _chunk(j, carry):
        pltpu.async_copy(g_ref.at[rowv.at[j]], buf, gsem).wait()

        def _scale(gi, c2):
            ew16 = ewv[j, pl.ds(gi * 16, 16)]
            for r in range(16):
                e = ew16[r]
                i = gi * 16 + r
                for k in range(8):
                    sl = pl.ds(k * 16, 16)
                    buf[i, sl] = buf[i, sl] * e
            return c2

        lax.fori_loop(0, _DCH // 16, _scale, 0)
        pltpu.sync_copy(buf, acc.at[colv.at[j]], add=True)
        return carry

    lax.fori_loop(0, _DNCH, _chunk, 0)
    plsc.subcore_barrier()
    pltpu.sync_copy(
        acc.at[pl.ds(sid * _RPT, _RPT)],
        out_ref.at[pl.ds(cid * N_PAD + sid * _RPT, _RPT)])


def _stage_d(g, rowd, cold, ewd, zeros):
    return pl.kernel(
        _d_body,
        out_type=jax.ShapeDtypeStruct((2 * N_PAD, HID), jnp.float32),
        mesh=plsc.VectorSubcoreMesh(core_axis_name="c", subcore_axis_name="s"),
        scratch_types=[
            pltpu.VMEM((_DNCH, _DCH), jnp.int32),
            pltpu.VMEM((_DNCH, _DCH), jnp.int32),
            pltpu.VMEM((_DNCH, _DCH), jnp.float32),
            pltpu.VMEM((_DCH, HID), jnp.float32),
            pltpu.VMEM_SHARED((N_PAD, HID), jnp.float32),
            pltpu.SemaphoreType.DMA,
        ],
    )(g, rowd, cold, ewd, zeros)


def _pad_edge_arr(a, padvals):
    return jnp.concatenate(
        [a.reshape(_NW, _EPT), padvals], axis=1).reshape(_NW * _DNCH, _DCH)


def _edge_weights_jax(pT, pPT, row, col):
    w = jnp.sum(pT[:, row] * pPT[:, col], axis=0)
    ssum = jnp.sum(w)
    ssq = jnp.sum(w * w)
    s1 = jax.ops.segment_sum(w, col, num_segments=N)
    cnt = jax.ops.segment_sum(jnp.ones((E,), jnp.float32), col, num_segments=N)
    return w, jnp.stack([ssum, ssq]), s1, cnt


def _conv_scatter_jax(g, row, col, ew):
    acc = jax.ops.segment_sum(ew[:, None] * g[row], col, num_segments=N_PAD)
    return jnp.concatenate([acc, jnp.zeros_like(acc)], axis=0)


def kernel(x, edge_index, W0, b0, W1, b1, W2, b2, l1W, l1b, l2W, l2b,
           mW1, mb1, mW2, mb2, mW3, mb3, parsing0):
    row, col = edge_index[0], edge_index[1]
    xp = jnp.pad(x, ((0, N_PAD - N), (0, 0)))

    pT, pPT, z0 = _stage_a(xp, mW1, mb1, mW2, mb2, mW3, mb3, parsing0, W0)

    w, stats, s1, cnt = _edge_weights_jax(pT, pPT, row, col)
    zpad = jnp.zeros((N_PAD - N,), jnp.float32)
    s1g = jnp.stack([jnp.concatenate([s1, zpad]),
                     jnp.zeros((N_PAD,), jnp.float32)], axis=1)
    cntg = jnp.stack([jnp.concatenate([cnt, zpad]),
                      jnp.zeros((N_PAD,), jnp.float32)], axis=1)

    dinv, g0, ab = _stage_c2(s1g, cntg, stats, z0)
    alpha, beta = ab[0, 0], ab[0, 1]
    ew = alpha * w + beta

    pad_r = ((jnp.arange(_NW * _EPAD, dtype=jnp.int32) * 97) % N
             ).reshape(_NW, _EPAD)
    pad_c = ((jnp.arange(_NW * _EPAD, dtype=jnp.int32) * 193 + 41) % N
             ).reshape(_NW, _EPAD)
    rowd = _pad_edge_arr(row, pad_r)
    cold = _pad_edge_arr(col, pad_c)
    ewd = _pad_edge_arr(ew, jnp.zeros((_NW, _EPAD), jnp.float32))

    zeros = jnp.zeros((_RPT, HID), jnp.float32)
    part0 = _stage_d(g0, rowd, cold, ewd, zeros)
    h0, g1 = _stage_e1(part0, g0, dinv, b0, W1)
    part1 = _stage_d(g1, rowd, cold, ewd, zeros)
    o = _stage_e2(part1, g1, dinv, b1, h0, l1W, l1b, l2W, l2b)
    return o[:N]


# speedup vs baseline: 17.4669x; 4.0351x over previous
import functools

import jax
import jax.numpy as jnp
from jax import lax
from jax.experimental import pallas as pl
from jax.experimental.pallas import tpu as pltpu
from jax.experimental.pallas import tpu_sc as plsc

N = 10000
E = 320000
D_IN = 128
HID = 128
OUT = 8

N_PAD = 10240
_NB = 4
_BN = N_PAD // _NB


def _a_body(x_ref, mW1_ref, mb1_ref, mW2_ref, mb2_ref, mW3_ref, mb3_ref,
            pars_ref, W0_ref, pT_ref, pPT_ref, z0_ref):
    xb = x_ref[...]
    h = jnp.maximum(xb @ mW1_ref[...] + mb1_ref[...], 0.0)
    h = jnp.maximum(h @ mW2_ref[...] + mb2_ref[...], 0.0)
    pT = lax.dot_general(mW3_ref[...], h, (((0,), (1,)), ((), ())),
                         preferred_element_type=jnp.float32)
    pT = pT + mb3_ref[...].reshape(OUT, 1)
    parsing = jnp.maximum(2.0 * pars_ref[...], 0.0)
    pPT = lax.dot_general(parsing, pT, (((0,), (0,)), ((), ())),
                          preferred_element_type=jnp.float32)
    pT_ref[...] = pT
    pPT_ref[...] = pPT
    z0_ref[...] = jnp.dot(xb, W0_ref[...], preferred_element_type=jnp.float32)


def _stage_a(x, mW1, mb1, mW2, mb2, mW3, mb3, parsing0, W0):
    full = lambda s: pl.BlockSpec(s, lambda i: (0, 0))
    return pl.pallas_call(
        _a_body,
        grid=(_NB,),
        in_specs=[
            pl.BlockSpec((_BN, D_IN), lambda i: (i, 0)),
            full((D_IN, 512)), full((1, 512)),
            full((512, 64)), full((1, 64)),
            full((64, OUT)), full((1, OUT)),
            full((OUT, OUT)),
            full((D_IN, HID)),
        ],
        out_specs=[
            pl.BlockSpec((OUT, _BN), lambda i: (0, i)),
            pl.BlockSpec((OUT, _BN), lambda i: (0, i)),
            pl.BlockSpec((_BN, HID), lambda i: (i, 0)),
        ],
        out_shape=[
            jax.ShapeDtypeStruct((OUT, N_PAD), jnp.float32),
            jax.ShapeDtypeStruct((OUT, N_PAD), jnp.float32),
            jax.ShapeDtypeStruct((_BN * _NB, HID), jnp.float32),
        ],
    )(x, mW1, mb1.reshape(1, 512), mW2, mb2.reshape(1, 64), mW3,
      mb3.reshape(1, OUT), parsing0, W0)


def _c2_body(s1_ref, cnt_ref, stats_ref, z0_ref, dinv_ref, g0_ref, ab_ref):
    ssum = stats_ref[0]
    ssq = stats_ref[1]
    mean = ssum / E
    var = (ssq - ssum * ssum / E) / (E - 1)
    alpha = jnp.sqrt(1e-4 / var)
    beta = 1.0 - mean * alpha
    s1 = s1_ref[:, 0:1] + s1_ref[:, 1:2]
    cnt = cnt_ref[:, 0:1] + cnt_ref[:, 1:2]
    deg = alpha * s1 + beta * cnt + 1.0
    ok = deg > 0.0
    dinv = jnp.where(ok, lax.rsqrt(jnp.where(ok, deg, 1.0)), 0.0)
    dinv_ref[...] = dinv
    g0_ref[...] = dinv * z0_ref[...]
    lane = lax.broadcasted_iota(jnp.int32, (1, 16), 1)
    ab_ref[...] = jnp.where(lane == 0, alpha,
                            jnp.where(lane == 1, beta, 0.0))


def _stage_c2(s1g, cntg, stats, z0):
    return pl.pallas_call(
        _c2_body,
        grid=(_NB,),
        in_specs=[
            pl.BlockSpec((_BN, 2), lambda i: (i, 0)),
            pl.BlockSpec((_BN, 2), lambda i: (i, 0)),
            pl.BlockSpec(memory_space=pltpu.SMEM),
            pl.BlockSpec((_BN, HID), lambda i: (i, 0)),
        ],
        out_specs=[
            pl.BlockSpec((_BN, 1), lambda i: (i, 0)),
            pl.BlockSpec((_BN, HID), lambda i: (i, 0)),
            pl.BlockSpec((1, 16), lambda i: (0, 0)),
        ],
        out_shape=[
            jax.ShapeDtypeStruct((N_PAD, 1), jnp.float32),
            jax.ShapeDtypeStruct((N_PAD, HID), jnp.float32),
            jax.ShapeDtypeStruct((1, 16), jnp.float32),
        ],
    )(s1g, cntg, stats, z0)


def _e1_body(pa_ref, pb_ref, g0_ref, dinv_ref, b0_ref, W1_ref,
             h0_ref, g1_ref):
    dinv = dinv_ref[...]
    h0 = jnp.maximum(
        dinv * (pa_ref[...] + pb_ref[...] + g0_ref[...]) + b0_ref[...], 0.0)
    h0_ref[...] = h0
    z1 = jnp.dot(h0, W1_ref[...], preferred_element_type=jnp.float32)
    g1_ref[...] = dinv * z1


def _stage_e1(part, g0, dinv, b0, W1):
    return pl.pallas_call(
        _e1_body,
        grid=(_NB,),
        in_specs=[
            pl.BlockSpec((_BN, HID), lambda i: (i, 0)),
            pl.BlockSpec((_BN, HID), lambda i: (i + _NB, 0)),
            pl.BlockSpec((_BN, HID), lambda i: (i, 0)),
            pl.BlockSpec((_BN, 1), lambda i: (i, 0)),
            pl.BlockSpec((1, HID), lambda i: (0, 0)),
            pl.BlockSpec((HID, HID), lambda i: (0, 0)),
        ],
        out_specs=[
            pl.BlockSpec((_BN, HID), lambda i: (i, 0)),
            pl.BlockSpec((_BN, HID), lambda i: (i, 0)),
        ],
        out_shape=[
            jax.ShapeDtypeStruct((N_PAD, HID), jnp.float32),
            jax.ShapeDtypeStruct((N_PAD, HID), jnp.float32),
        ],
    )(part, part, g0, dinv, b0.reshape(1, HID), W1)


def _e2_body(pa_ref, pb_ref, g1_ref, dinv_ref, b1_ref, h0_ref,
             l1a_ref, l1b_ref, l1bias_ref, l2W_ref, l2b_ref, o_ref):
    dinv = dinv_ref[...]
    h1 = jnp.maximum(
        dinv * (pa_ref[...] + pb_ref[...] + g1_ref[...]) + b1_ref[...], 0.0)
    j = (jnp.dot(h0_ref[...], l1a_ref[...], preferred_element_type=jnp.float32)
         + jnp.dot(h1, l1b_ref[...], preferred_element_type=jnp.float32)
         + l1bias_ref[...])
    j = jnp.maximum(j, 0.0)
    o_ref[...] = (jnp.dot(j, l2W_ref[...], preferred_element_type=jnp.float32)
                  + l2b_ref[...])


def _stage_e2(part, g1, dinv, b1, h0, l1W, l1b, l2W, l2b):
    return pl.pallas_call(
        _e2_body,
        grid=(_NB,),
        in_specs=[
            pl.BlockSpec((_BN, HID), lambda i: (i, 0)),
            pl.BlockSpec((_BN, HID), lambda i: (i + _NB, 0)),
            pl.BlockSpec((_BN, HID), lambda i: (i, 0)),
            pl.BlockSpec((_BN, 1), lambda i: (i, 0)),
            pl.BlockSpec((1, HID), lambda i: (0, 0)),
            pl.BlockSpec((_BN, HID), lambda i: (i, 0)),
            pl.BlockSpec((HID, HID), lambda i: (0, 0)),
            pl.BlockSpec((HID, HID), lambda i: (0, 0)),
            pl.BlockSpec((1, HID), lambda i: (0, 0)),
            pl.BlockSpec((HID, OUT), lambda i: (0, 0)),
            pl.BlockSpec((1, OUT), lambda i: (0, 0)),
        ],
        out_specs=pl.BlockSpec((_BN, OUT), lambda i: (i, 0)),
        out_shape=jax.ShapeDtypeStruct((N_PAD, OUT), jnp.float32),
    )(part, part, g1, dinv, b1.reshape(1, HID), h0, l1W[:HID], l1W[HID:],
      l1b.reshape(1, HID), l2W, l2b.reshape(1, OUT))


_BEPT = E // 16
_BNCH = 160
_BPAD = _BNCH * 128 - _BEPT
_BSC = 40
_BSUP = _BNCH // _BSC


def _b_body(ptab_ref, qtab_ref, rowb_ref, colb_ref, onesb_ref, z1_ref,
            wout_ref, s1out_ref, cntout_ref,
            ptv, qtv, rowv, colv, onesv, wbuf, s1sp, cntsp):
    cid = lax.axis_index("c")
    sid = lax.axis_index("s")
    pltpu.sync_copy(ptab_ref.at[pl.ds(cid * 4 * N_PAD, 4 * N_PAD)], ptv)
    pltpu.sync_copy(qtab_ref.at[pl.ds(cid * 4 * N_PAD, 4 * N_PAD)], qtv)
    pltpu.sync_copy(z1_ref.at[pl.ds(sid * 640, 640)],
                    s1sp.at[pl.ds(sid * 640, 640)])
    pltpu.sync_copy(z1_ref.at[pl.ds(sid * 640, 640)],
                    cntsp.at[pl.ds(sid * 640, 640)])
    plsc.subcore_barrier()

    base = sid * _BNCH

    def _sup(b, carry):
        off = base + b * _BSC
        pltpu.sync_copy(rowb_ref.at[pl.ds(off, _BSC)], rowv)
        pltpu.sync_copy(colb_ref.at[pl.ds(off, _BSC)], colv)
        pltpu.sync_copy(onesb_ref.at[pl.ds(off, _BSC)], onesv)

        def _chunk(jj, c2):
            for g in range(8):
                sl = pl.ds(g * 16, 16)
                r16 = rowv[jj, sl]
                c16 = colv[jj, sl]
                acc = None
                for d in range(4):
                    sp = plsc.load_gather(ptv, [r16 + d * N_PAD])
                    ep = plsc.load_gather(qtv, [c16 + d * N_PAD])
                    acc = sp * ep if acc is None else acc + sp * ep
                wbuf[jj, sl] = acc * onesv[jj, sl]
            pltpu.sync_copy(wbuf.at[jj], s1sp.at[colv.at[jj]], add=True)

            @pl.when(cid == 0)
            def _cnt():
                pltpu.sync_copy(onesv.at[jj], cntsp.at[colv.at[jj]],
                                add=True)

            return c2

        lax.fori_loop(0, _BSC, _chunk, 0)
        pltpu.sync_copy(wbuf, wout_ref.at[pl.ds(cid * 2560 + off, _BSC)])
        return carry

    lax.fori_loop(0, _BSUP, _sup, 0)
    plsc.subcore_barrier()
    pltpu.sync_copy(s1sp.at[pl.ds(sid * 640, 640)],
                    s1out_ref.at[pl.ds(cid * N_PAD + sid * 640, 640)])
    pltpu.sync_copy(cntsp.at[pl.ds(sid * 640, 640)],
                    cntout_ref.at[pl.ds(cid * N_PAD + sid * 640, 640)])


def _stage_b(ptab, qtab, rowb, colb, onesb, zeros1):
    return pl.kernel(
        _b_body,
        out_type=[
            jax.ShapeDtypeStruct((5120, 128), jnp.float32),
            jax.ShapeDtypeStruct((2 * N_PAD,), jnp.float32),
            jax.ShapeDtypeStruct((2 * N_PAD,), jnp.float32),
        ],
        mesh=plsc.VectorSubcoreMesh(core_axis_name="c", subcore_axis_name="s"),
        compiler_params=pltpu.CompilerParams(needs_layout_passes=False),
        scratch_types=[
            pltpu.VMEM((4 * N_PAD,), jnp.float32),
            pltpu.VMEM((4 * N_PAD,), jnp.float32),
            pltpu.VMEM((_BSC, 128), jnp.int32),
            pltpu.VMEM((_BSC, 128), jnp.int32),
            pltpu.VMEM((_BSC, 128), jnp.float32),
            pltpu.VMEM((_BSC, 128), jnp.float32),
            pltpu.VMEM_SHARED((N_PAD,), jnp.float32),
            pltpu.VMEM_SHARED((N_PAD,), jnp.float32),
        ],
    )(ptab, qtab, rowb, colb, onesb, zeros1)


def _c1_body(wa_ref, wb_ref, wsum_ref, stats_ref, acc_ref):
    i = pl.program_id(0)
    w = wa_ref[0] + wb_ref[0]
    wsum_ref[...] = w

    @pl.when(i == 0)
    def _init():
        acc_ref[0] = 0.0
        acc_ref[1] = 0.0

    acc_ref[0] += jnp.sum(w)
    acc_ref[1] += jnp.sum(w * w)

    @pl.when(i == 3)
    def _fin():
        stats_ref[0] = acc_ref[0]
        stats_ref[1] = acc_ref[1]


def _stage_c1(wout3):
    return pl.pallas_call(
        _c1_body,
        grid=(4,),
        in_specs=[
            pl.BlockSpec((1, 640, 128), lambda i: (0, i, 0)),
            pl.BlockSpec((1, 640, 128), lambda i: (1, i, 0)),
        ],
        out_specs=[
            pl.BlockSpec((640, 128), lambda i: (i, 0)),
            pl.BlockSpec(memory_space=pltpu.SMEM),
        ],
        out_shape=[
            jax.ShapeDtypeStruct((2560, 128), jnp.float32),
            jax.ShapeDtypeStruct((2,), jnp.float32),
        ],
        scratch_shapes=[pltpu.SMEM((2,), jnp.float32)],
    )(wout3, wout3)


def _c1b_body(w_ref, ab_ref, ew_ref):
    ew_ref[...] = ab_ref[0] * w_ref[...] + ab_ref[1]


def _stage_c1b(wsum, ab16):
    return pl.pallas_call(
        _c1b_body,
        grid=(4,),
        in_specs=[
            pl.BlockSpec((640, 128), lambda i: (i, 0)),
            pl.BlockSpec(memory_space=pltpu.SMEM),
        ],
        out_specs=pl.BlockSpec((640, 128), lambda i: (i, 0)),
        out_shape=jax.ShapeDtypeStruct((2560, 128), jnp.float32),
    )(wsum, ab16)


_NW = 32
_EPT = E // _NW
_DCH = 128
_DNCH = 80
_EPAD = _DNCH * _DCH - _EPT
_RPT = N_PAD // 16


def _d_body(g_ref, row_ref, col_ref, ew_ref, zeros_ref, out_ref,
            rowv, colv, ewv, buf, acc, gsem):
    cid = lax.axis_index("c")
    sid = lax.axis_index("s")
    wid = sid * 2 + cid
    base = wid * _DNCH
    pltpu.sync_copy(row_ref.at[pl.ds(base, _DNCH)], rowv)
    pltpu.sync_copy(col_ref.at[pl.ds(base, _DNCH)], colv)
    pltpu.sync_copy(ew_ref.at[pl.ds(base, _DNCH)], ewv)

    pltpu.sync_copy(zeros_ref, acc.at[pl.ds(sid * _RPT, _RPT)])
    plsc.subcore_barrier()

    def _chunk(j, carry):
        pltpu.async_copy(g_ref.at[rowv.at[j]], buf, gsem).wait()

        def _scale(gi, c2):
            ew16 = ewv[j, pl.ds(gi * 16, 16)]
            for r in range(16):
                e = ew16[r]
                i = gi * 16 + r
                for k in range(8):
                    sl = pl.ds(k * 16, 16)
                    buf[i, sl] = buf[i, sl] * e
            return c2

        lax.fori_loop(0, _DCH // 16, _scale, 0)
        pltpu.sync_copy(buf, acc.at[colv.at[j]], add=True)
        return carry

    lax.fori_loop(0, _DNCH, _chunk, 0)
    plsc.subcore_barrier()
    pltpu.sync_copy(
        acc.at[pl.ds(sid * _RPT, _RPT)],
        out_ref.at[pl.ds(cid * N_PAD + sid * _RPT, _RPT)])


def _stage_d(g, rowd, cold, ewd, zeros):
    return pl.kernel(
        _d_body,
        out_type=jax.ShapeDtypeStruct((2 * N_PAD, HID), jnp.float32),
        mesh=plsc.VectorSubcoreMesh(core_axis_name="c", subcore_axis_name="s"),
        scratch_types=[
            pltpu.VMEM((_DNCH, _DCH), jnp.int32),
            pltpu.VMEM((_DNCH, _DCH), jnp.int32),
            pltpu.VMEM((_DNCH, _DCH), jnp.float32),
            pltpu.VMEM((_DCH, HID), jnp.float32),
            pltpu.VMEM_SHARED((N_PAD, HID), jnp.float32),
            pltpu.SemaphoreType.DMA,
        ],
    )(g, rowd, cold, ewd, zeros)


def _pad_edge_arr(a, padvals):
    return jnp.concatenate(
        [a.reshape(_NW, _EPT), padvals], axis=1).reshape(_NW * _DNCH, _DCH)


def _pad_b_arr(a, padvals):
    return jnp.concatenate(
        [a.reshape(16, _BEPT), padvals], axis=1).reshape(16 * _BNCH, 128)


def kernel(x, edge_index, W0, b0, W1, b1, W2, b2, l1W, l1b, l2W, l2b,
           mW1, mb1, mW2, mb2, mW3, mb3, parsing0):
    row, col = edge_index[0], edge_index[1]
    xp = jnp.pad(x, ((0, N_PAD - N), (0, 0)))

    ptab, qtab, z0 = _stage_a(xp, mW1, mb1, mW2, mb2, mW3, mb3, parsing0, W0)

    padb_r = ((jnp.arange(16 * _BPAD, dtype=jnp.int32) * 97) % N
              ).reshape(16, _BPAD)
    padb_c = ((jnp.arange(16 * _BPAD, dtype=jnp.int32) * 193 + 41) % N
              ).reshape(16, _BPAD)
    rowb = _pad_b_arr(row, padb_r)
    colb = _pad_b_arr(col, padb_c)
    onesb = _pad_b_arr(jnp.ones((E,), jnp.float32),
                       jnp.zeros((16, _BPAD), jnp.float32))
    zeros1 = jnp.zeros((N_PAD,), jnp.float32)

    wout, s1o, cnto = _stage_b(ptab.reshape(OUT * N_PAD),
                               qtab.reshape(OUT * N_PAD),
                               rowb, colb, onesb, zeros1)
    wsum, stats = _stage_c1(wout.reshape(2, 2560, 128))
    s1g = s1o.reshape(2, N_PAD).T
    cntg = cnto.reshape(2, N_PAD).T

    dinv, g0, ab = _stage_c2(s1g, cntg, stats, z0)
    ewB = _stage_c1b(wsum, ab.reshape(16))
    ew = ewB.reshape(16, _BNCH * 128)[:, :_BEPT].reshape(E)

    pad_r = ((jnp.arange(_NW * _EPAD, dtype=jnp.int32) * 97) % N
             ).reshape(_NW, _EPAD)
    pad_c = ((jnp.arange(_NW * _EPAD, dtype=jnp.int32) * 193 + 41) % N
             ).reshape(_NW, _EPAD)
    rowd = _pad_edge_arr(row, pad_r)
    cold = _pad_edge_arr(col, pad_c)
    ewd = _pad_edge_arr(ew, jnp.zeros((_NW, _EPAD), jnp.float32))

    zeros = jnp.zeros((_RPT, HID), jnp.float32)
    part0 = _stage_d(g0, rowd, cold, ewd, zeros)
    h0, g1 = _stage_e1(part0, g0, dinv, b0, W1)
    part1 = _stage_d(g1, rowd, cold, ewd, zeros)
    o = _stage_e2(part1, g1, dinv, b1, h0, l1W, l1b, l2W, l2b)
    return o[:N]


# speedup vs baseline: 18.4048x; 1.0537x over previous
import functools

import jax
import jax.numpy as jnp
from jax import lax
from jax.experimental import pallas as pl
from jax.experimental.pallas import tpu as pltpu
from jax.experimental.pallas import tpu_sc as plsc

N = 10000
E = 320000
D_IN = 128
HID = 128
OUT = 8

N_PAD = 10240
_NB = 4
_BN = N_PAD // _NB


def _a_body(x_ref, mW1_ref, mb1_ref, mW2_ref, mb2_ref, mW3_ref, mb3_ref,
            pars_ref, W0_ref, pT_ref, pPT_ref, z0_ref):
    xb = x_ref[...]
    h = jnp.maximum(xb @ mW1_ref[...] + mb1_ref[...], 0.0)
    h = jnp.maximum(h @ mW2_ref[...] + mb2_ref[...], 0.0)
    pT = lax.dot_general(mW3_ref[...], h, (((0,), (1,)), ((), ())),
                         preferred_element_type=jnp.float32)
    pT = pT + mb3_ref[...].reshape(OUT, 1)
    parsing = jnp.maximum(2.0 * pars_ref[...], 0.0)
    pPT = lax.dot_general(parsing, pT, (((0,), (0,)), ((), ())),
                          preferred_element_type=jnp.float32)
    pT_ref[...] = pT
    pPT_ref[...] = pPT
    z0_ref[...] = jnp.dot(xb, W0_ref[...], preferred_element_type=jnp.float32)


def _stage_a(x, mW1, mb1, mW2, mb2, mW3, mb3, parsing0, W0):
    full = lambda s: pl.BlockSpec(s, lambda i: (0, 0))
    return pl.pallas_call(
        _a_body,
        grid=(_NB,),
        in_specs=[
            pl.BlockSpec((_BN, D_IN), lambda i: (i, 0)),
            full((D_IN, 512)), full((1, 512)),
            full((512, 64)), full((1, 64)),
            full((64, OUT)), full((1, OUT)),
            full((OUT, OUT)),
            full((D_IN, HID)),
        ],
        out_specs=[
            pl.BlockSpec((OUT, _BN), lambda i: (0, i)),
            pl.BlockSpec((OUT, _BN), lambda i: (0, i)),
            pl.BlockSpec((_BN, HID), lambda i: (i, 0)),
        ],
        out_shape=[
            jax.ShapeDtypeStruct((OUT, N_PAD), jnp.float32),
            jax.ShapeDtypeStruct((OUT, N_PAD), jnp.float32),
            jax.ShapeDtypeStruct((_BN * _NB, HID), jnp.float32),
        ],
    )(x, mW1, mb1.reshape(1, 512), mW2, mb2.reshape(1, 64), mW3,
      mb3.reshape(1, OUT), parsing0, W0)


def _c2_body(s1_ref, cnt_ref, stats_ref, z0_ref, dinv_ref, g0_ref, ab_ref):
    ssum = stats_ref[0]
    ssq = stats_ref[1]
    mean = ssum / E
    var = (ssq - ssum * ssum / E) / (E - 1)
    alpha = jnp.sqrt(1e-4 / var)
    beta = 1.0 - mean * alpha
    s1 = s1_ref[:, 0:1] + s1_ref[:, 1:2]
    cnt = cnt_ref[:, 0:1] + cnt_ref[:, 1:2]
    deg = alpha * s1 + beta * cnt + 1.0
    ok = deg > 0.0
    dinv = jnp.where(ok, lax.rsqrt(jnp.where(ok, deg, 1.0)), 0.0)
    dinv_ref[...] = dinv
    g0_ref[...] = dinv * z0_ref[...]
    lane = lax.broadcasted_iota(jnp.int32, (1, 16), 1)
    ab_ref[...] = jnp.where(lane == 0, alpha,
                            jnp.where(lane == 1, beta, 0.0))


def _stage_c2(s1g, cntg, stats, z0):
    return pl.pallas_call(
        _c2_body,
        grid=(_NB,),
        in_specs=[
            pl.BlockSpec((_BN, 2), lambda i: (i, 0)),
            pl.BlockSpec((_BN, 2), lambda i: (i, 0)),
            pl.BlockSpec(memory_space=pltpu.SMEM),
            pl.BlockSpec((_BN, HID), lambda i: (i, 0)),
        ],
        out_specs=[
            pl.BlockSpec((_BN, 1), lambda i: (i, 0)),
            pl.BlockSpec((_BN, HID), lambda i: (i, 0)),
            pl.BlockSpec((1, 16), lambda i: (0, 0)),
        ],
        out_shape=[
            jax.ShapeDtypeStruct((N_PAD, 1), jnp.float32),
            jax.ShapeDtypeStruct((N_PAD, HID), jnp.float32),
            jax.ShapeDtypeStruct((1, 16), jnp.float32),
        ],
    )(s1g, cntg, stats, z0)


def _e1_body(pa_ref, pb_ref, g0_ref, dinv_ref, b0_ref, W1_ref,
             h0_ref, g1_ref):
    dinv = dinv_ref[...]
    h0 = jnp.maximum(
        dinv * (pa_ref[...] + pb_ref[...] + g0_ref[...]) + b0_ref[...], 0.0)
    h0_ref[...] = h0
    z1 = jnp.dot(h0, W1_ref[...], preferred_element_type=jnp.float32)
    g1_ref[...] = dinv * z1


def _stage_e1(part, g0, dinv, b0, W1):
    return pl.pallas_call(
        _e1_body,
        grid=(_NB,),
        in_specs=[
            pl.BlockSpec((_BN, HID), lambda i: (i, 0)),
            pl.BlockSpec((_BN, HID), lambda i: (i + _NB, 0)),
            pl.BlockSpec((_BN, HID), lambda i: (i, 0)),
            pl.BlockSpec((_BN, 1), lambda i: (i, 0)),
            pl.BlockSpec((1, HID), lambda i: (0, 0)),
            pl.BlockSpec((HID, HID), lambda i: (0, 0)),
        ],
        out_specs=[
            pl.BlockSpec((_BN, HID), lambda i: (i, 0)),
            pl.BlockSpec((_BN, HID), lambda i: (i, 0)),
        ],
        out_shape=[
            jax.ShapeDtypeStruct((N_PAD, HID), jnp.float32),
            jax.ShapeDtypeStruct((N_PAD, HID), jnp.float32),
        ],
    )(part, part, g0, dinv, b0.reshape(1, HID), W1)


def _e2_body(pa_ref, pb_ref, g1_ref, dinv_ref, b1_ref, h0_ref,
             l1a_ref, l1b_ref, l1bias_ref, l2W_ref, l2b_ref, o_ref):
    dinv = dinv_ref[...]
    h1 = jnp.maximum(
        dinv * (pa_ref[...] + pb_ref[...] + g1_ref[...]) + b1_ref[...], 0.0)
    j = (jnp.dot(h0_ref[...], l1a_ref[...], preferred_element_type=jnp.float32)
         + jnp.dot(h1, l1b_ref[...], preferred_element_type=jnp.float32)
         + l1bias_ref[...])
    j = jnp.maximum(j, 0.0)
    o_ref[...] = (jnp.dot(j, l2W_ref[...], preferred_element_type=jnp.float32)
                  + l2b_ref[...])


def _stage_e2(part, g1, dinv, b1, h0, l1W, l1b, l2W, l2b):
    return pl.pallas_call(
        _e2_body,
        grid=(_NB,),
        in_specs=[
            pl.BlockSpec((_BN, HID), lambda i: (i, 0)),
            pl.BlockSpec((_BN, HID), lambda i: (i + _NB, 0)),
            pl.BlockSpec((_BN, HID), lambda i: (i, 0)),
            pl.BlockSpec((_BN, 1), lambda i: (i, 0)),
            pl.BlockSpec((1, HID), lambda i: (0, 0)),
            pl.BlockSpec((_BN, HID), lambda i: (i, 0)),
            pl.BlockSpec((HID, HID), lambda i: (0, 0)),
            pl.BlockSpec((HID, HID), lambda i: (0, 0)),
            pl.BlockSpec((1, HID), lambda i: (0, 0)),
            pl.BlockSpec((HID, OUT), lambda i: (0, 0)),
            pl.BlockSpec((1, OUT), lambda i: (0, 0)),
        ],
        out_specs=pl.BlockSpec((_BN, OUT), lambda i: (i, 0)),
        out_shape=jax.ShapeDtypeStruct((N_PAD, OUT), jnp.float32),
    )(part, part, g1, dinv, b1.reshape(1, HID), h0, l1W[:HID], l1W[HID:],
      l1b.reshape(1, HID), l2W, l2b.reshape(1, OUT))


_BEPT = E // 16
_BNCH = 160
_BPAD = _BNCH * 128 - _BEPT
_BSC = 40
_BSUP = _BNCH // _BSC


def _b_body(ptab_ref, qtab_ref, rowb_ref, colb_ref, onesb_ref, z1_ref,
            wout_ref, s1out_ref, cntout_ref,
            ptv, qtv, rowv, colv, onesv, wbuf, s1sp, cntsp, ssem, csem):
    cid = lax.axis_index("c")
    sid = lax.axis_index("s")
    pltpu.sync_copy(ptab_ref.at[pl.ds(cid * 4 * N_PAD, 4 * N_PAD)], ptv)
    pltpu.sync_copy(qtab_ref.at[pl.ds(cid * 4 * N_PAD, 4 * N_PAD)], qtv)
    pltpu.sync_copy(z1_ref.at[pl.ds(sid * 640, 640)],
                    s1sp.at[pl.ds(sid * 640, 640)])
    pltpu.sync_copy(z1_ref.at[pl.ds(sid * 640, 640)],
                    cntsp.at[pl.ds(sid * 640, 640)])
    plsc.subcore_barrier()

    base = sid * _BNCH

    def _sup(b, carry):
        off = base + b * _BSC
        pltpu.sync_copy(rowb_ref.at[pl.ds(off, _BSC)], rowv)
        pltpu.sync_copy(colb_ref.at[pl.ds(off, _BSC)], colv)
        pltpu.sync_copy(onesb_ref.at[pl.ds(off, _BSC)], onesv)

        def _chunk(jj, c2):
            for g in range(8):
                sl = pl.ds(g * 16, 16)
                r16 = rowv[jj, sl]
                c16 = colv[jj, sl]
                acc = None
                for d in range(4):
                    sp = plsc.load_gather(ptv, [r16 + d * N_PAD])
                    ep = plsc.load_gather(qtv, [c16 + d * N_PAD])
                    acc = sp * ep if acc is None else acc + sp * ep
                wbuf[jj, sl] = acc * onesv[jj, sl]
            pltpu.async_copy(wbuf.at[jj], s1sp.at[colv.at[jj]], ssem,
                             add=True)

            @pl.when(cid == 0)
            def _cnt():
                pltpu.async_copy(onesv.at[jj], cntsp.at[colv.at[jj]], csem,
                                 add=True)

            return c2

        lax.fori_loop(0, _BSC, _chunk, 0)

        def _drain(jj, c2):
            pltpu.make_async_copy(wbuf.at[jj], s1sp.at[colv.at[jj]],
                                  ssem).wait()

            @pl.when(cid == 0)
            def _dcnt():
                pltpu.make_async_copy(onesv.at[jj], cntsp.at[colv.at[jj]],
                                      csem).wait()

            return c2

        lax.fori_loop(0, _BSC, _drain, 0)
        pltpu.sync_copy(wbuf, wout_ref.at[pl.ds(cid * 2560 + off, _BSC)])
        return carry

    lax.fori_loop(0, _BSUP, _sup, 0)
    plsc.subcore_barrier()
    pltpu.sync_copy(s1sp.at[pl.ds(sid * 640, 640)],
                    s1out_ref.at[pl.ds(cid * N_PAD + sid * 640, 640)])
    pltpu.sync_copy(cntsp.at[pl.ds(sid * 640, 640)],
                    cntout_ref.at[pl.ds(cid * N_PAD + sid * 640, 640)])


def _stage_b(ptab, qtab, rowb, colb, onesb, zeros1):
    return pl.kernel(
        _b_body,
        out_type=[
            jax.ShapeDtypeStruct((5120, 128), jnp.float32),
            jax.ShapeDtypeStruct((2 * N_PAD,), jnp.float32),
            jax.ShapeDtypeStruct((2 * N_PAD,), jnp.float32),
        ],
        mesh=plsc.VectorSubcoreMesh(core_axis_name="c", subcore_axis_name="s"),
        compiler_params=pltpu.CompilerParams(needs_layout_passes=False),
        scratch_types=[
            pltpu.VMEM((4 * N_PAD,), jnp.float32),
            pltpu.VMEM((4 * N_PAD,), jnp.float32),
            pltpu.VMEM((_BSC, 128), jnp.int32),
            pltpu.VMEM((_BSC, 128), jnp.int32),
            pltpu.VMEM((_BSC, 128), jnp.float32),
            pltpu.VMEM((_BSC, 128), jnp.float32),
            pltpu.VMEM_SHARED((N_PAD,), jnp.float32),
            pltpu.VMEM_SHARED((N_PAD,), jnp.float32),
            pltpu.SemaphoreType.DMA,
            pltpu.SemaphoreType.DMA,
        ],
    )(ptab, qtab, rowb, colb, onesb, zeros1)


def _c1_body(wa_ref, wb_ref, wsum_ref, stats_ref, acc_ref):
    i = pl.program_id(0)
    w = wa_ref[0] + wb_ref[0]
    wsum_ref[...] = w

    @pl.when(i == 0)
    def _init():
        acc_ref[0] = 0.0
        acc_ref[1] = 0.0

    acc_ref[0] += jnp.sum(w)
    acc_ref[1] += jnp.sum(w * w)

    @pl.when(i == 3)
    def _fin():
        stats_ref[0] = acc_ref[0]
        stats_ref[1] = acc_ref[1]


def _stage_c1(wout3):
    return pl.pallas_call(
        _c1_body,
        grid=(4,),
        in_specs=[
            pl.BlockSpec((1, 640, 128), lambda i: (0, i, 0)),
            pl.BlockSpec((1, 640, 128), lambda i: (1, i, 0)),
        ],
        out_specs=[
            pl.BlockSpec((640, 128), lambda i: (i, 0)),
            pl.BlockSpec(memory_space=pltpu.SMEM),
        ],
        out_shape=[
            jax.ShapeDtypeStruct((2560, 128), jnp.float32),
            jax.ShapeDtypeStruct((2,), jnp.float32),
        ],
        scratch_shapes=[pltpu.SMEM((2,), jnp.float32)],
    )(wout3, wout3)


def _c1b_body(w_ref, ones_ref, ab_ref, ew_ref):
    ew_ref[...] = (ab_ref[0] * w_ref[...] + ab_ref[1]) * ones_ref[...]


def _stage_c1b(wsum, onesb, ab16):
    return pl.pallas_call(
        _c1b_body,
        grid=(4,),
        in_specs=[
            pl.BlockSpec((640, 128), lambda i: (i, 0)),
            pl.BlockSpec((640, 128), lambda i: (i, 0)),
            pl.BlockSpec(memory_space=pltpu.SMEM),
        ],
        out_specs=pl.BlockSpec((640, 128), lambda i: (i, 0)),
        out_shape=jax.ShapeDtypeStruct((2560, 128), jnp.float32),
    )(wsum, onesb, ab16)


_NW = 32
_EPT = E // _NW
_DCH = 128
_DNCH = 80
_EPAD = _DNCH * _DCH - _EPT
_RPT = N_PAD // 16


def _d_body(g_ref, row_ref, col_ref, ew_ref, zeros_ref, out_ref,
            rowv, colv, ewv, buf, acc, gsem):
    cid = lax.axis_index("c")
    sid = lax.axis_index("s")
    base = sid * (2 * _DNCH) + cid * _DNCH
    pltpu.sync_copy(row_ref.at[pl.ds(base, _DNCH)], rowv)
    pltpu.sync_copy(col_ref.at[pl.ds(base, _DNCH)], colv)
    pltpu.sync_copy(ew_ref.at[pl.ds(base, _DNCH)], ewv)

    pltpu.sync_copy(zeros_ref, acc.at[pl.ds(sid * _RPT, _RPT)])
    plsc.subcore_barrier()

    def _chunk(j, carry):
        pltpu.async_copy(g_ref.at[rowv.at[j]], buf, gsem).wait()

        def _scale(gi, c2):
            ew16 = ewv[j, pl.ds(gi * 16, 16)]
            for r in range(16):
                e = ew16[r]
                i = gi * 16 + r
                for k in range(8):
                    sl = pl.ds(k * 16, 16)
                    buf[i, sl] = buf[i, sl] * e
            return c2

        lax.fori_loop(0, _DCH // 16, _scale, 0)
        pltpu.sync_copy(buf, acc.at[colv.at[j]], add=True)
        return carry

    lax.fori_loop(0, _DNCH, _chunk, 0)
    plsc.subcore_barrier()
    pltpu.sync_copy(
        acc.at[pl.ds(sid * _RPT, _RPT)],
        out_ref.at[pl.ds(cid * N_PAD + sid * _RPT, _RPT)])


def _stage_d(g, rowd, cold, ewd, zeros):
    return pl.kernel(
        _d_body,
        out_type=jax.ShapeDtypeStruct((2 * N_PAD, HID), jnp.float32),
        mesh=plsc.VectorSubcoreMesh(core_axis_name="c", subcore_axis_name="s"),
        scratch_types=[
            pltpu.VMEM((_DNCH, _DCH), jnp.int32),
            pltpu.VMEM((_DNCH, _DCH), jnp.int32),
            pltpu.VMEM((_DNCH, _DCH), jnp.float32),
            pltpu.VMEM((_DCH, HID), jnp.float32),
            pltpu.VMEM_SHARED((N_PAD, HID), jnp.float32),
            pltpu.SemaphoreType.DMA,
        ],
    )(g, rowd, cold, ewd, zeros)


def _pad_edge_arr(a, padvals):
    return jnp.concatenate(
        [a.reshape(_NW, _EPT), padvals], axis=1).reshape(_NW * _DNCH, _DCH)


def _pad_b_arr(a, padvals):
    return jnp.concatenate(
        [a.reshape(16, _BEPT), padvals], axis=1).reshape(16 * _BNCH, 128)


def kernel(x, edge_index, W0, b0, W1, b1, W2, b2, l1W, l1b, l2W, l2b,
           mW1, mb1, mW2, mb2, mW3, mb3, parsing0):
    row, col = edge_index[0], edge_index[1]
    xp = jnp.pad(x, ((0, N_PAD - N), (0, 0)))

    ptab, qtab, z0 = _stage_a(xp, mW1, mb1, mW2, mb2, mW3, mb3, parsing0, W0)

    padb_r = ((jnp.arange(16 * _BPAD, dtype=jnp.int32) * 97) % N
              ).reshape(16, _BPAD)
    padb_c = ((jnp.arange(16 * _BPAD, dtype=jnp.int32) * 193 + 41) % N
              ).reshape(16, _BPAD)
    rowb = _pad_b_arr(row, padb_r)
    colb = _pad_b_arr(col, padb_c)
    onesb = _pad_b_arr(jnp.ones((E,), jnp.float32),
                       jnp.zeros((16, _BPAD), jnp.float32))
    zeros1 = jnp.zeros((N_PAD,), jnp.float32)

    wout, s1o, cnto = _stage_b(ptab.reshape(OUT * N_PAD),
                               qtab.reshape(OUT * N_PAD),
                               rowb, colb, onesb, zeros1)
    wsum, stats = _stage_c1(wout.reshape(2, 2560, 128))
    s1g = s1o.reshape(2, N_PAD).T
    cntg = cnto.reshape(2, N_PAD).T

    dinv, g0, ab = _stage_c2(s1g, cntg, stats, z0)
    ewB = _stage_c1b(wsum, onesb, ab.reshape(16))

    zeros = jnp.zeros((_RPT, HID), jnp.float32)
    part0 = _stage_d(g0, rowb, colb, ewB, zeros)
    h0, g1 = _stage_e1(part0, g0, dinv, b0, W1)
    part1 = _stage_d(g1, rowb, colb, ewB, zeros)
    o = _stage_e2(part1, g1, dinv, b1, h0, l1W, l1b, l2W, l2b)
    return o[:N]
